# baseline (speedup 1.0000x reference)
import jax
import jax.numpy as jnp
from jax.experimental import pallas as pl
from jax.experimental.pallas import tpu as pltpu

T = 64
D = 1024
E = 64
DFF = 1024
HALF = E // 2


def _routing_step(gate_ref, w_ref, sched_ref):
    g = gate_ref[...]
    idx = jax.lax.broadcasted_iota(jnp.int32, (T, E), 1)
    m1 = jnp.max(g, axis=-1, keepdims=True)
    a1 = jnp.min(jnp.where(g == m1, idx, E), axis=-1, keepdims=True)
    g2 = jnp.where(idx == a1, -jnp.inf, g)
    m2 = jnp.max(g2, axis=-1, keepdims=True)
    a2 = jnp.min(jnp.where(g2 == m2, idx, E), axis=-1, keepdims=True)
    p2 = jnp.exp(m2 - m1)
    w1 = 1.0 / (1.0 + p2)
    w2 = p2 / (1.0 + p2)
    w = jnp.where(idx == a1, w1, 0.0) + jnp.where(idx == a2, w2, 0.0)
    w_ref[...] = w

    active = (jnp.sum(jnp.where(w > 0.0, 1, 0), axis=0, keepdims=True) > 0)
    active_i = active.astype(jnp.int32)
    je = jax.lax.broadcasted_iota(jnp.int32, (E, E), 0)
    ee = jax.lax.broadcasted_iota(jnp.int32, (E, E), 1)
    pos = jnp.sum(jnp.where(je <= ee, active_i[0][:, None], 0), axis=0)[None, :] - 1
    n = jnp.sum(active_i)
    lane = jax.lax.broadcasted_iota(jnp.int32, (1, E), 1)
    onehot = (pos[0][None, :] == lane[0][:, None]) & active[0][None, :]
    order = jnp.sum(jnp.where(onehot, lane[0][None, :], 0), axis=1)[None, :]

    h = (n + 1) // 2
    idx0 = jnp.minimum(lane, h - 1)
    idx1 = h + jnp.minimum(lane, jnp.maximum(n - h - 1, 0))
    idx1 = jnp.minimum(idx1, E - 1)
    gath0 = (ee == idx0[0][:, None])
    gath1 = (ee == idx1[0][:, None])
    order0 = jnp.sum(jnp.where(gath0, order[0][None, :], 0), axis=1)[None, :]
    order1 = jnp.sum(jnp.where(gath1, order[0][None, :], 0), axis=1)[None, :]
    valid0 = (lane < h).astype(jnp.int32)
    valid1 = (lane < (n - h)).astype(jnp.int32)
    sched_ref[...] = jnp.concatenate(
        [order0, valid0, order1, valid1, jnp.zeros((4, E), jnp.int32)], axis=0)


def _moe_step(sched_ref, x_ref, w_ref, gup_ref, dp_ref, out_ref):
    i = pl.program_id(1)

    @pl.when(i == 0)
    def _init():
        out_ref[...] = jnp.zeros_like(out_ref)

    x = x_ref[...]
    w1e = gup_ref[0]
    gu = jax.lax.dot_general(
        x, w1e, (((1,), (1,)), ((), ())),
        preferred_element_type=jnp.float32)
    gate = gu[:, :DFF]
    up = gu[:, DFF:]
    h = gate * jax.nn.sigmoid(gate) * up
    oe = jax.lax.dot_general(
        h, dp_ref[0], (((1,), (1,)), ((), ())),
        preferred_element_type=jnp.float32)
    c = pl.program_id(0)
    e_id = sched_ref[2 * c, i]
    scale = sched_ref[2 * c + 1, i].astype(jnp.float32)
    eidx = jax.lax.broadcasted_iota(jnp.int32, (T, E), 1)
    we = jnp.sum(jnp.where(eidx == e_id, w_ref[...], 0.0), axis=1, keepdims=True)
    out_ref[0] += (scale * we) * oe


@jax.jit
def kernel(x, gating_output, gate_up_proj, down_proj):
    w_te, sched = pl.pallas_call(
        _routing_step,
        in_specs=[pl.BlockSpec((T, E), lambda: (0, 0))],
        out_specs=[
            pl.BlockSpec((T, E), lambda: (0, 0)),
            pl.BlockSpec((8, E), lambda: (0, 0)),
        ],
        out_shape=[
            jax.ShapeDtypeStruct((T, E), jnp.float32),
            jax.ShapeDtypeStruct((8, E), jnp.int32),
        ],
    )(gating_output)

    parts = pl.pallas_call(
        _moe_step,
        grid_spec=pltpu.PrefetchScalarGridSpec(
            num_scalar_prefetch=1,
            grid=(2, HALF),
            in_specs=[
                pl.BlockSpec((T, D), lambda c, i, s: (0, 0)),
                pl.BlockSpec((T, E), lambda c, i, s: (0, 0)),
                pl.BlockSpec((1, 2 * DFF, D), lambda c, i, s: (s[2 * c, i], 0, 0)),
                pl.BlockSpec((1, D, DFF), lambda c, i, s: (s[2 * c, i], 0, 0)),
            ],
            out_specs=pl.BlockSpec((1, T, D), lambda c, i, s: (c, 0, 0)),
        ),
        out_shape=jax.ShapeDtypeStruct((2, T, D), jnp.float32),
        compiler_params=pltpu.CompilerParams(
            dimension_semantics=("parallel", "arbitrary")),
    )(sched, x, w_te, gate_up_proj, down_proj)
    return parts[0] + parts[1]


# speedup vs baseline: 1.0123x; 1.0123x over previous
import jax
import jax.numpy as jnp
from jax.experimental import pallas as pl
from jax.experimental.pallas import tpu as pltpu

T = 64
D = 1024
E = 64
DFF = 1024


def _routing_step(gate_ref, w_ref, sched_ref):
    g = gate_ref[...]
    idx = jax.lax.broadcasted_iota(jnp.int32, (T, E), 1)
    m1 = jnp.max(g, axis=-1, keepdims=True)
    a1 = jnp.min(jnp.where(g == m1, idx, E), axis=-1, keepdims=True)
    g2 = jnp.where(idx == a1, -jnp.inf, g)
    m2 = jnp.max(g2, axis=-1, keepdims=True)
    a2 = jnp.min(jnp.where(g2 == m2, idx, E), axis=-1, keepdims=True)
    p2 = jnp.exp(m2 - m1)
    w1 = 1.0 / (1.0 + p2)
    w2 = p2 / (1.0 + p2)
    w = jnp.where(idx == a1, w1, 0.0) + jnp.where(idx == a2, w2, 0.0)
    w_ref[...] = w

    active = (jnp.sum(jnp.where(w > 0.0, 1, 0), axis=0, keepdims=True) > 0)
    active_i = active.astype(jnp.int32)
    je = jax.lax.broadcasted_iota(jnp.int32, (E, E), 0)
    ee = jax.lax.broadcasted_iota(jnp.int32, (E, E), 1)
    pos = jnp.sum(jnp.where(je <= ee, active_i[0][:, None], 0), axis=0)[None, :] - 1
    num_active = jnp.sum(active_i)
    lane = jax.lax.broadcasted_iota(jnp.int32, (1, E), 1)
    onehot = (pos[0][None, :] == lane[0][:, None]) & active[0][None, :]
    order = jnp.sum(jnp.where(onehot, lane[0][None, :], 0), axis=1)[None, :]
    last = jnp.sum(jnp.where((pos == num_active - 1) & active, lane, 0))
    valid = (lane < num_active)
    order = jnp.where(valid, order, last)
    sched_ref[...] = jnp.concatenate(
        [order, valid.astype(jnp.int32), jnp.zeros((6, E), jnp.int32)], axis=0)


def _moe_step(sched_ref, x_ref, w_ref, gp_ref, up_ref, dp_ref, out_ref):
    i = pl.program_id(0)

    @pl.when(i == 0)
    def _init():
        out_ref[...] = jnp.zeros_like(out_ref)

    x = x_ref[...]
    gate = jax.lax.dot_general(
        x, gp_ref[0, 0], (((1,), (1,)), ((), ())),
        preferred_element_type=jnp.float32)
    up = jax.lax.dot_general(
        x, up_ref[0, 0], (((1,), (1,)), ((), ())),
        preferred_element_type=jnp.float32)
    h = gate * jax.nn.sigmoid(gate) * up
    oe = jax.lax.dot_general(
        h, dp_ref[0], (((1,), (1,)), ((), ())),
        preferred_element_type=jnp.float32)
    e_id = sched_ref[0, i]
    scale = sched_ref[1, i].astype(jnp.float32)
    eidx = jax.lax.broadcasted_iota(jnp.int32, (T, E), 1)
    we = jnp.sum(jnp.where(eidx == e_id, w_ref[...], 0.0), axis=1, keepdims=True)
    out_ref[...] += (scale * we) * oe


@jax.jit
def kernel(x, gating_output, gate_up_proj, down_proj):
    w_te, sched = pl.pallas_call(
        _routing_step,
        in_specs=[pl.BlockSpec((T, E), lambda: (0, 0))],
        out_specs=[
            pl.BlockSpec((T, E), lambda: (0, 0)),
            pl.BlockSpec((8, E), lambda: (0, 0)),
        ],
        out_shape=[
            jax.ShapeDtypeStruct((T, E), jnp.float32),
            jax.ShapeDtypeStruct((8, E), jnp.int32),
        ],
    )(gating_output)

    gup4 = gate_up_proj.reshape(E, 2, DFF, D)
    return pl.pallas_call(
        _moe_step,
        grid_spec=pltpu.PrefetchScalarGridSpec(
            num_scalar_prefetch=1,
            grid=(E,),
            in_specs=[
                pl.BlockSpec((T, D), lambda i, s: (0, 0)),
                pl.BlockSpec((T, E), lambda i, s: (0, 0)),
                pl.BlockSpec((1, 1, DFF, D), lambda i, s: (s[0, i], 0, 0, 0)),
                pl.BlockSpec((1, 1, DFF, D), lambda i, s: (s[0, i], 1, 0, 0)),
                pl.BlockSpec((1, D, DFF), lambda i, s: (s[0, i], 0, 0)),
            ],
            out_specs=pl.BlockSpec((T, D), lambda i, s: (0, 0)),
        ),
        out_shape=jax.ShapeDtypeStruct((T, D), jnp.float32),
    )(sched, x, w_te, gup4, gup4, down_proj)


# speedup vs baseline: 1.1026x; 1.0892x over previous
import jax
import jax.numpy as jnp
from jax.experimental import pallas as pl
from jax.experimental.pallas import tpu as pltpu

T = 64
D = 1024
E = 64
DFF = 1024


def _routing_step(gate_ref, w_ref, sched_ref):
    g = gate_ref[...]
    idx = jax.lax.broadcasted_iota(jnp.int32, (T, E), 1)
    m1 = jnp.max(g, axis=-1, keepdims=True)
    a1 = jnp.min(jnp.where(g == m1, idx, E), axis=-1, keepdims=True)
    g2 = jnp.where(idx == a1, -jnp.inf, g)
    m2 = jnp.max(g2, axis=-1, keepdims=True)
    a2 = jnp.min(jnp.where(g2 == m2, idx, E), axis=-1, keepdims=True)
    p2 = jnp.exp(m2 - m1)
    w1 = 1.0 / (1.0 + p2)
    w2 = p2 / (1.0 + p2)
    w = jnp.where(idx == a1, w1, 0.0) + jnp.where(idx == a2, w2, 0.0)
    w_ref[...] = w

    active = (jnp.sum(jnp.where(w > 0.0, 1, 0), axis=0, keepdims=True) > 0)
    active_i = active.astype(jnp.int32)
    je = jax.lax.broadcasted_iota(jnp.int32, (E, E), 0)
    ee = jax.lax.broadcasted_iota(jnp.int32, (E, E), 1)
    pos = jnp.sum(jnp.where(je <= ee, active_i[0][:, None], 0), axis=0)[None, :] - 1
    num_active = jnp.sum(active_i)
    lane = jax.lax.broadcasted_iota(jnp.int32, (1, E), 1)
    onehot = (pos[0][None, :] == lane[0][:, None]) & active[0][None, :]
    order = jnp.sum(jnp.where(onehot, lane[0][None, :], 0), axis=1)[None, :]
    last = jnp.sum(jnp.where((pos == num_active - 1) & active, lane, 0))
    valid = (lane < num_active)
    order = jnp.where(valid, order, last)
    sched_ref[...] = jnp.concatenate(
        [order, valid.astype(jnp.int32), jnp.zeros((6, E), jnp.int32)], axis=0)


def _moe_step(sched_ref, x_ref, w_ref, gp_ref, up_ref, dp_ref, out_ref):
    i = pl.program_id(0)

    @pl.when(i == 0)
    def _init():
        out_ref[...] = jnp.zeros_like(out_ref)

    @pl.when(sched_ref[1, i] == 1)
    def _compute():
        x = x_ref[...]
        gate = jax.lax.dot_general(
            x, gp_ref[0, 0], (((1,), (1,)), ((), ())),
            preferred_element_type=jnp.float32)
        up = jax.lax.dot_general(
            x, up_ref[0, 0], (((1,), (1,)), ((), ())),
            preferred_element_type=jnp.float32)
        h = gate * jax.nn.sigmoid(gate) * up
        oe = jax.lax.dot_general(
            h, dp_ref[0], (((1,), (1,)), ((), ())),
            preferred_element_type=jnp.float32)
        e_id = sched_ref[0, i]
        eidx = jax.lax.broadcasted_iota(jnp.int32, (T, E), 1)
        we = jnp.sum(jnp.where(eidx == e_id, w_ref[...], 0.0), axis=1, keepdims=True)
        out_ref[...] += we * oe


@jax.jit
def kernel(x, gating_output, gate_up_proj, down_proj):
    w_te, sched = pl.pallas_call(
        _routing_step,
        in_specs=[pl.BlockSpec((T, E), lambda: (0, 0))],
        out_specs=[
            pl.BlockSpec((T, E), lambda: (0, 0)),
            pl.BlockSpec((8, E), lambda: (0, 0)),
        ],
        out_shape=[
            jax.ShapeDtypeStruct((T, E), jnp.float32),
            jax.ShapeDtypeStruct((8, E), jnp.int32),
        ],
    )(gating_output)

    gup4 = gate_up_proj.reshape(E, 2, DFF, D)
    return pl.pallas_call(
        _moe_step,
        grid_spec=pltpu.PrefetchScalarGridSpec(
            num_scalar_prefetch=1,
            grid=(E,),
            in_specs=[
                pl.BlockSpec((T, D), lambda i, s: (0, 0)),
                pl.BlockSpec((T, E), lambda i, s: (0, 0)),
                pl.BlockSpec((1, 1, DFF, D), lambda i, s: (s[0, i], 0, 0, 0)),
                pl.BlockSpec((1, 1, DFF, D), lambda i, s: (s[0, i], 1, 0, 0)),
                pl.BlockSpec((1, D, DFF), lambda i, s: (s[0, i], 0, 0)),
            ],
            out_specs=pl.BlockSpec((T, D), lambda i, s: (0, 0)),
        ),
        out_shape=jax.ShapeDtypeStruct((T, D), jnp.float32),
    )(sched, x, w_te, gup4, gup4, down_proj)
